# baseline (speedup 1.0000x reference)
import functools

import jax
import jax.numpy as jnp
from jax import lax
from jax.experimental import pallas as pl
from jax.experimental.pallas import tpu as pltpu
from jax.experimental.pallas import tpu_sc as plsc

_NUM_CORES = 2
_NUM_SUBCORES = 16
_NW = _NUM_CORES * _NUM_SUBCORES


def kernel(tokens, W_pos):
    batch, seq = tokens.shape
    d = W_pos.shape[1]
    rows_per_w = seq // _NW
    chunks = (48, 48, 32)

    mesh = plsc.VectorSubcoreMesh(core_axis_name="c", subcore_axis_name="s")

    @functools.partial(
        pl.kernel,
        mesh=mesh,
        out_type=jax.ShapeDtypeStruct((batch, seq, d), W_pos.dtype),
        scratch_types=[
            pltpu.VMEM((max(chunks), d), W_pos.dtype),
            pltpu.SemaphoreType.DMA,
        ],
    )
    def _copy(w_hbm, out_hbm, buf, sem):
        wid = lax.axis_index("s") * _NUM_CORES + lax.axis_index("c")
        base = wid * rows_per_w
        off = 0
        for chunk in chunks:
            start = base + off
            off += chunk
            pltpu.sync_copy(w_hbm.at[pl.ds(start, chunk), :],
                            buf.at[pl.ds(0, chunk), :])
            handles = [
                pltpu.async_copy(buf.at[pl.ds(0, chunk), :],
                                 out_hbm.at[b, pl.ds(start, chunk), :], sem)
                for b in range(batch)
            ]
            for h in handles:
                h.wait()

    return _copy(W_pos)
